# baseline (speedup 1.0000x reference)
import functools

import jax
import jax.numpy as jnp
from jax import lax
from jax.experimental import pallas as pl
from jax.experimental.pallas import tpu as pltpu
from jax.experimental.pallas import tpu_sc as plsc

N = 50000
NPAD = 51200
RPT = NPAD // 16
E = 800000
EPAD = 819200
PADN = 50016
NBLK = 400
B = 1024
CNT_BINS = 2048
CNT_PT = CNT_BINS // 16
POOL_ROWS = 1152
POOL_PT = POOL_ROWS // 16

RB = 3200
D2R = RB // 128
GRID = 16
EPS = 1e-5
PREC = jax.lax.Precision.HIGHEST

@functools.lru_cache(maxsize=None)
def _sc_mesh():
    return plsc.VectorSubcoreMesh(core_axis_name="c", subcore_axis_name="s",
                                  num_cores=2, num_subcores=16)


def _sc_deg_cnt(dst_flat, batch2d, zrpt, ones1k):
    @functools.partial(
        pl.kernel,
        out_type=(jax.ShapeDtypeStruct((2 * NPAD,), jnp.float32),
                  jax.ShapeDtypeStruct((2 * CNT_BINS,), jnp.float32)),
        mesh=_sc_mesh(),
        compiler_params=pltpu.CompilerParams(use_tc_tiling_on_sc=False),
        scratch_types=[
            pltpu.VMEM_SHARED((NPAD,), jnp.float32),
            pltpu.VMEM_SHARED((CNT_BINS,), jnp.float32),
            pltpu.VMEM((1024,), jnp.int32),
            pltpu.VMEM((16, 128), jnp.int32),
            pltpu.VMEM((1024,), jnp.float32),
        ],
    )
    def k(dst_hbm, batch_hbm, z_hbm, ones_hbm, deg_out, cnt_out,
          sh_deg, sh_cnt, didx_v, bidx_v, ones_v):
        c = lax.axis_index("c")
        s = lax.axis_index("s")
        pltpu.sync_copy(ones_hbm, ones_v)
        pltpu.sync_copy(z_hbm, sh_deg.at[pl.ds(s * RPT, RPT)])
        pltpu.sync_copy(z_hbm.at[pl.ds(0, CNT_PT)],
                        sh_cnt.at[pl.ds(s * CNT_PT, CNT_PT)])
        ebase = c * 409600 + s * 25600
        w = c * 16 + s
        nj = 8 + 8 * (w < 18).astype(jnp.int32)

        @pl.when(w < 18)
        def _():
            pltpu.sync_copy(batch_hbm.at[pl.ds(w * 16, 16)], bidx_v)

        @pl.when(w >= 18)
        def _():
            pltpu.sync_copy(batch_hbm.at[pl.ds(288 + (w - 18) * 8, 8)],
                            bidx_v.at[pl.ds(0, 8)])

        plsc.subcore_barrier()

        @pl.loop(0, 25)
        def _(j):
            pltpu.sync_copy(dst_hbm.at[pl.ds(ebase + j * 1024, 1024)],
                            didx_v)
            pltpu.sync_copy(ones_v, sh_deg.at[didx_v], add=True)

        @pl.loop(0, nj)
        def _(j):
            pltpu.sync_copy(ones_v.at[pl.ds(0, 128)],
                            sh_cnt.at[bidx_v.at[j]], add=True)

        plsc.subcore_barrier()
        pltpu.sync_copy(sh_deg.at[pl.ds(s * RPT, RPT)],
                        deg_out.at[pl.ds(c * NPAD + s * RPT, RPT)])
        pltpu.sync_copy(sh_cnt.at[pl.ds(s * CNT_PT, CNT_PT)],
                        cnt_out.at[pl.ds(c * CNT_BINS + s * CNT_PT, CNT_PT)])

    return k(dst_flat, batch2d, zrpt, ones1k)


def _sc_prop(ms, src_flat, dst_flat, zinit, CT, SBE=256):
    CPH = CT // 2
    split_edges = CT == 1
    SB = (25600 if split_edges else 51200) // SBE
    OUTC = 2 if split_edges else CT

    @functools.partial(
        pl.kernel,
        out_type=jax.ShapeDtypeStruct((OUTC, NPAD, 32), jnp.float32),
        mesh=_sc_mesh(),
        compiler_params=pltpu.CompilerParams(use_tc_tiling_on_sc=False),
        scratch_types=[
            pltpu.VMEM_SHARED((NPAD, 32), jnp.float32),
            pltpu.VMEM((SBE,), jnp.int32),
            pltpu.VMEM((SBE,), jnp.int32),
            pltpu.VMEM((SBE,), jnp.int32),
            pltpu.VMEM((SBE,), jnp.int32),
            pltpu.VMEM((SBE, 32), jnp.float32),
            pltpu.VMEM((SBE, 32), jnp.float32),
            pltpu.SemaphoreType.DMA,
            pltpu.SemaphoreType.DMA,
            pltpu.SemaphoreType.DMA,
            pltpu.SemaphoreType.DMA,
        ],
    )
    def k(ms_hbm, src_hbm, dst_hbm, z_hbm, out_hbm,
          sh, srcv0, srcv1, dstv0, dstv1, buf0, buf1,
          gsem0, gsem1, ssem0, ssem1):
        c = lax.axis_index("c")
        s = lax.axis_index("s")
        srcv = (srcv0, srcv1)
        dstv = (dstv0, dstv1)
        bufs = (buf0, buf1)
        gsems = (gsem0, gsem1)
        ssems = (ssem0, ssem1)
        if split_edges:
            ebase = c * 409600 + s * 25600
        else:
            ebase = s * 51200

        def load_idx(b, sb):
            pltpu.sync_copy(src_hbm.at[pl.ds(ebase + sb * SBE, SBE)],
                            srcv[b])
            pltpu.sync_copy(dst_hbm.at[pl.ds(ebase + sb * SBE, SBE)],
                            dstv[b])

        def do_edges(src2d):
            load_idx(0, 0)
            pltpu.async_copy(src2d.at[srcv0], buf0, gsem0)

            @pl.loop(0, SB, step=2)
            def _(j):
                for b in range(2):
                    sb = j + b
                    b2 = 1 - b
                    pltpu.make_async_copy(src2d.at[srcv[b]],
                                          bufs[b], gsems[b]).wait()
                    pltpu.async_copy(bufs[b], sh.at[dstv[b]], ssems[b],
                                     add=True)

                    @pl.when(sb >= 1)
                    def _():
                        pltpu.make_async_copy(bufs[b2], sh.at[dstv[b2]],
                                              ssems[b2]).wait()

                    @pl.when(sb + 1 < SB)
                    def _():
                        load_idx(b2, sb + 1)
                        pltpu.async_copy(src2d.at[srcv[b2]],
                                         bufs[b2], gsems[b2])

            pltpu.make_async_copy(bufs[1], sh.at[dstv1], ssems[1]).wait()
            plsc.subcore_barrier()

        if split_edges:
            @pl.when(c == 0)
            def _():
                pltpu.sync_copy(ms_hbm.at[pl.ds(s * RPT, RPT)],
                                sh.at[pl.ds(s * RPT, RPT)])

            @pl.when(c == 1)
            def _():
                pltpu.sync_copy(z_hbm, sh.at[pl.ds(s * RPT, RPT)])

            plsc.subcore_barrier()
            do_edges(ms_hbm)
            pltpu.sync_copy(sh.at[pl.ds(s * RPT, RPT)],
                            out_hbm.at[c].at[pl.ds(s * RPT, RPT)])
        else:
            def do_chunk(ci):
                pltpu.sync_copy(ms_hbm.at[ci].at[pl.ds(s * RPT, RPT)],
                                sh.at[pl.ds(s * RPT, RPT)])
                plsc.subcore_barrier()
                do_edges(ms_hbm.at[ci])
                pltpu.sync_copy(sh.at[pl.ds(s * RPT, RPT)],
                                out_hbm.at[ci].at[pl.ds(s * RPT, RPT)])

            @pl.when(c == 0)
            def _():
                for kk in range(CPH):
                    do_chunk(kk)

            @pl.when(c == 1)
            def _():
                for kk in range(CPH):
                    do_chunk(CPH + kk)

    return k(ms, src_flat, dst_flat, zinit)


def _sc_pool(gx, batch2d, zpool):
    @functools.partial(
        pl.kernel,
        out_type=jax.ShapeDtypeStruct((2, POOL_ROWS, 128), jnp.float32),
        mesh=_sc_mesh(),
        compiler_params=pltpu.CompilerParams(use_tc_tiling_on_sc=False),
        scratch_types=[
            pltpu.VMEM_SHARED((POOL_ROWS, 128), jnp.float32),
            pltpu.VMEM((16, 128), jnp.int32),
            pltpu.VMEM((128, 128), jnp.float32),
        ],
    )
    def k(gx_hbm, batch_hbm, z_hbm, out_hbm, sh, bidx, buf):
        c = lax.axis_index("c")
        s = lax.axis_index("s")
        w = c * 16 + s
        base_blk = jnp.where(w < 18, w * 16, 288 + (w - 18) * 8)
        nj = 8 + 8 * (w < 18).astype(jnp.int32)

        @pl.when(w < 18)
        def _():
            pltpu.sync_copy(batch_hbm.at[pl.ds(w * 16, 16)], bidx)

        @pl.when(w >= 18)
        def _():
            pltpu.sync_copy(batch_hbm.at[pl.ds(288 + (w - 18) * 8, 8)],
                            bidx.at[pl.ds(0, 8)])

        pltpu.sync_copy(z_hbm, sh.at[pl.ds(s * POOL_PT, POOL_PT)])
        plsc.subcore_barrier()

        @pl.loop(0, nj)
        def _(j):
            pltpu.sync_copy(gx_hbm.at[pl.ds((base_blk + j) * 128, 128)], buf)
            pltpu.sync_copy(buf, sh.at[bidx.at[j]], add=True)

        plsc.subcore_barrier()
        pltpu.sync_copy(sh.at[pl.ds(s * POOL_PT, POOL_PT)],
                        out_hbm.at[c].at[pl.ds(s * POOL_PT, POOL_PT)])

    return k(gx, batch2d, zpool)


def _row_mask(i, rows):
    rid = lax.broadcasted_iota(jnp.int32, (rows, 1), 0) + i * rows
    return (rid < N).astype(jnp.float32)


def _tc1_body(x_ref, dp_ref, ms_ref, dis_ref):
    deg = dp_ref[0] + dp_ref[1] + 1.0
    dis = lax.rsqrt(deg)
    ms_ref[...] = x_ref[...] * dis
    dis_ref[...] = dis


def _tc1(xpad, deg_part):
    return pl.pallas_call(
        _tc1_body,
        grid=(GRID,),
        in_specs=[pl.BlockSpec((RB, 32), lambda i: (i, 0)),
                  pl.BlockSpec((2, RB, 1), lambda i: (0, i, 0))],
        out_specs=[pl.BlockSpec((RB, 32), lambda i: (i, 0)),
                   pl.BlockSpec((RB, 1), lambda i: (i, 0))],
        out_shape=[jax.ShapeDtypeStruct((NPAD, 32), jnp.float32),
                   jax.ShapeDtypeStruct((NPAD, 1), jnp.float32)],
    )(xpad, deg_part)


def _tc2a_body(p1_ref, dis_ref, w_ref, b_ref, a1_ref, st_ref):
    i = pl.program_id(0)
    prop = (p1_ref[0] + p1_ref[1]) * dis_ref[...]
    a1 = jnp.dot(prop, w_ref[...], preferred_element_type=jnp.float32,
                 precision=PREC) + b_ref[...][None, :]
    a1_ref[...] = a1
    am = a1 * _row_mask(i, RB)
    s1 = jnp.sum(am, axis=0)
    s2 = jnp.sum(am * am, axis=0)

    @pl.when(i == 0)
    def _():
        st_ref[...] = jnp.zeros_like(st_ref)

    st_ref[...] += jnp.concatenate(
        [s1[None, :], s2[None, :], jnp.zeros((6, s1.shape[0]), jnp.float32)], axis=0)


def _tc2a(p1, dis2d, Wg1, bg1):
    return pl.pallas_call(
        _tc2a_body,
        grid=(GRID,),
        in_specs=[pl.BlockSpec((2, RB, 32), lambda i: (0, i, 0)),
                  pl.BlockSpec((RB, 1), lambda i: (i, 0)),
                  pl.BlockSpec((32, 256), lambda i: (0, 0)),
                  pl.BlockSpec((256,), lambda i: (0,))],
        out_specs=[pl.BlockSpec((RB, 256), lambda i: (i, 0)),
                   pl.BlockSpec((8, 256), lambda i: (0, 0))],
        out_shape=[jax.ShapeDtypeStruct((NPAD, 256), jnp.float32),
                   jax.ShapeDtypeStruct((8, 256), jnp.float32)],
    )(p1, dis2d, Wg1, bg1)


def _tc2b_body(a1_ref, st_ref, g_ref, bta_ref, w2_ref, dis_ref, ms2_ref):
    i = pl.program_id(0)
    st = st_ref[...]
    mu = st[0, :] / N
    var = st[1, :] / N - mu * mu
    inv = lax.rsqrt(var + EPS) * g_ref[...]
    h = jnp.maximum((a1_ref[...] - mu[None, :]) * inv[None, :]
                    + bta_ref[...][None, :], 0.0)
    m2 = jnp.dot(h, w2_ref[...], preferred_element_type=jnp.float32,
                 precision=PREC)
    ms2 = m2 * dis_ref[...] * _row_mask(i, RB)
    ms2_ref[...] = jnp.stack(
        [ms2[:, 32 * k:32 * (k + 1)] for k in range(4)], axis=0)


def _tc2b(a1, stats1, gamma1, beta1, Wg2, dis2d):
    return pl.pallas_call(
        _tc2b_body,
        grid=(GRID,),
        in_specs=[pl.BlockSpec((RB, 256), lambda i: (i, 0)),
                  pl.BlockSpec((8, 256), lambda i: (0, 0)),
                  pl.BlockSpec((256,), lambda i: (0,)),
                  pl.BlockSpec((256,), lambda i: (0,)),
                  pl.BlockSpec((256, 128), lambda i: (0, 0)),
                  pl.BlockSpec((RB, 1), lambda i: (i, 0))],
        out_specs=pl.BlockSpec((4, RB, 32), lambda i: (0, i, 0)),
        out_shape=jax.ShapeDtypeStruct((4, NPAD, 32), jnp.float32),
    )(a1, stats1, gamma1, beta1, Wg2, dis2d)


def _tc3a_body(p2_ref, dis_ref, b_ref, a2_ref, st_ref):
    i = pl.program_id(0)
    a2 = jnp.concatenate([p2_ref[k] for k in range(4)],
                         axis=1) * dis_ref[...] + b_ref[...][None, :]
    a2_ref[...] = a2
    am = a2 * _row_mask(i, RB)
    s1 = jnp.sum(am, axis=0)
    s2 = jnp.sum(am * am, axis=0)

    @pl.when(i == 0)
    def _():
        st_ref[...] = jnp.zeros_like(st_ref)

    st_ref[...] += jnp.concatenate(
        [s1[None, :], s2[None, :], jnp.zeros((6, s1.shape[0]), jnp.float32)], axis=0)


def _tc3a(p2, dis2d, bg2):
    return pl.pallas_call(
        _tc3a_body,
        grid=(GRID,),
        in_specs=[pl.BlockSpec((4, RB, 32), lambda i: (0, i, 0)),
                  pl.BlockSpec((RB, 1), lambda i: (i, 0)),
                  pl.BlockSpec((128,), lambda i: (0,))],
        out_specs=[pl.BlockSpec((RB, 128), lambda i: (i, 0)),
                   pl.BlockSpec((8, 128), lambda i: (0, 0))],
        out_shape=[jax.ShapeDtypeStruct((NPAD, 128), jnp.float32),
                   jax.ShapeDtypeStruct((8, 128), jnp.float32)],
    )(p2, dis2d, bg2)


def _tc3b_body(a2_ref, st_ref, g_ref, bta_ref, gx_ref):
    i = pl.program_id(0)
    st = st_ref[...]
    mu = st[0, :] / N
    var = st[1, :] / N - mu * mu
    inv = lax.rsqrt(var + EPS) * g_ref[...]
    gx = jnp.maximum((a2_ref[...] - mu[None, :]) * inv[None, :]
                     + bta_ref[...][None, :], 0.0)
    gx_ref[...] = gx * _row_mask(i, RB)


def _tc3b(a2, stats2, gamma2, beta2):
    return pl.pallas_call(
        _tc3b_body,
        grid=(GRID,),
        in_specs=[pl.BlockSpec((RB, 128), lambda i: (i, 0)),
                  pl.BlockSpec((8, 128), lambda i: (0, 0)),
                  pl.BlockSpec((128,), lambda i: (0,)),
                  pl.BlockSpec((128,), lambda i: (0,))],
        out_specs=pl.BlockSpec((RB, 128), lambda i: (i, 0)),
        out_shape=jax.ShapeDtypeStruct((NPAD, 128), jnp.float32),
    )(a2, stats2, gamma2, beta2)


def _tc_codetab_body(cd_ref, td_ref, wc1, bc1, wc2, bc2, wc3, bc3,
                     wt1, bt1, wt2, bt2, co_ref, to_ref):
    def mm(a, w, b):
        return jnp.dot(a, w[...], preferred_element_type=jnp.float32,
                       precision=PREC) + b[...][None, :]

    h = jnp.maximum(mm(cd_ref[...], wc1, bc1), 0.0)
    h = jnp.maximum(mm(h, wc2, bc2), 0.0)
    co_ref[...] = mm(h, wc3, bc3)
    t = jnp.maximum(mm(td_ref[...], wt1, bt1), 0.0)
    to_ref[...] = jnp.maximum(mm(t, wt2, bt2), 0.0)


def _tc_codetab(code_data, tabular_data, Wc1, bc1, Wc2, bc2, Wc3, bc3,
                Wt1, bt1, Wt2, bt2):
    return pl.pallas_call(
        _tc_codetab_body,
        out_shape=[jax.ShapeDtypeStruct((B, 32), jnp.float32),
                   jax.ShapeDtypeStruct((B, 32), jnp.float32)],
    )(code_data, tabular_data, Wc1, bc1, Wc2, bc2, Wc3, bc3,
      Wt1, bt1, Wt2, bt2)


def _tc4_body(co_ref, to_ref, pp_ref, cp_ref, mw_ref,
              wf1, bf1, wf2, bf2, wf3, bf3, out_ref):
    mwrow = mw_ref[0, :]
    lane = lax.broadcasted_iota(jnp.int32, (128,), 0)
    valid = lane < 3
    mx = jnp.max(jnp.where(valid, mwrow, -jnp.inf))
    e = jnp.where(valid, jnp.exp(mwrow - mx), 0.0)
    wts = e / jnp.sum(e)
    cnt = cp_ref[0, :B] + cp_ref[1, :B]
    pooled = (pp_ref[0, :B, :] + pp_ref[1, :B, :]) \
        / jnp.maximum(cnt, 1.0)[:, None]
    fused = jnp.concatenate(
        [co_ref[...] * wts[0], to_ref[...] * wts[1], pooled * wts[2]], axis=1)

    def mm(a, w, b):
        return jnp.dot(a, w[...], preferred_element_type=jnp.float32,
                       precision=PREC) + b[...][None, :]

    o = jnp.maximum(mm(fused, wf1, bf1), 0.0)
    o = jnp.maximum(mm(o, wf2, bf2), 0.0)
    out_ref[...] = mm(o, wf3, bf3)


def _tc4(code_out, tab_out, pool_part, cnt_part, mwp,
         Wf1, bf1, Wf2, bf2, Wf3, bf3):
    return pl.pallas_call(
        _tc4_body,
        out_shape=jax.ShapeDtypeStruct((B, 4), jnp.float32),
    )(code_out, tab_out, pool_part, cnt_part, mwp,
      Wf1, bf1, Wf2, bf2, Wf3, bf3)


def kernel(code_data, tabular_data, x, edge_index, batch,
           Wc1, bc1, Wc2, bc2, Wc3, bc3,
           Wt1, bt1, Wt2, bt2,
           Wg1, bg1, gamma1, beta1, Wg2, bg2, gamma2, beta2,
           mw, Wf1, bf1, Wf2, bf2, Wf3, bf3):
    epad = jnp.full((EPAD - E,), PADN, jnp.int32)
    src_flat = jnp.concatenate([edge_index[0], epad])
    dst_flat = jnp.concatenate([edge_index[1], epad])
    batch2d = jnp.concatenate(
        [batch, jnp.full((NPAD - N,), B, jnp.int32)]).reshape(NBLK, 128)
    xpad = jnp.pad(x, ((0, NPAD - N), (0, 0)))
    zrpt = jnp.zeros((RPT,), jnp.float32)
    ones1k = jnp.ones((1024,), jnp.float32)
    zinit32 = jnp.zeros((RPT, 32), jnp.float32)
    zpool = jnp.zeros((POOL_PT, 128), jnp.float32)
    mwp = jnp.zeros((1, 128), jnp.float32).at[0, :3].set(mw)

    code_out, tab_out = _tc_codetab(code_data, tabular_data,
                                    Wc1, bc1, Wc2, bc2, Wc3, bc3,
                                    Wt1, bt1, Wt2, bt2)

    deg_flat, cnt_flat = _sc_deg_cnt(dst_flat, batch2d, zrpt, ones1k)
    deg_part = deg_flat.reshape(2, NPAD, 1)
    cnt_part = cnt_flat.reshape(2, CNT_BINS)
    ms1, dis2d = _tc1(xpad, deg_part)
    p1 = _sc_prop(ms1, src_flat, dst_flat, zinit32, CT=1)
    a1, stats1 = _tc2a(p1, dis2d, Wg1, bg1)
    ms2 = _tc2b(a1, stats1, gamma1, beta1, Wg2, dis2d)
    p2 = _sc_prop(ms2, src_flat, dst_flat, zinit32, CT=4)
    a2, stats2 = _tc3a(p2, dis2d, bg2)
    gx = _tc3b(a2, stats2, gamma2, beta2)
    pool_part = _sc_pool(gx, batch2d, zpool)

    return _tc4(code_out, tab_out, pool_part, cnt_part, mwp,
                Wf1, bf1, Wf2, bf2, Wf3, bf3)


# speedup vs baseline: 1.2742x; 1.2742x over previous
import functools

import jax
import jax.numpy as jnp
from jax import lax
from jax.experimental import pallas as pl
from jax.experimental.pallas import tpu as pltpu
from jax.experimental.pallas import tpu_sc as plsc

N = 50000
NPAD = 51200
RPT = NPAD // 16
E = 800000
EPAD = 819200
PADN = 50016
NBLK = 400
B = 1024
CNT_BINS = 2048
CNT_PT = CNT_BINS // 16
POOL_ROWS = 1152
POOL_PT = POOL_ROWS // 16

RB = 3200
D2R = RB // 128
GRID = 16
EPS = 1e-5
PREC = jax.lax.Precision.HIGHEST

@functools.lru_cache(maxsize=None)
def _sc_mesh():
    return plsc.VectorSubcoreMesh(core_axis_name="c", subcore_axis_name="s",
                                  num_cores=2, num_subcores=16)


def _sc_deg_cnt(dst_flat, batch2d, zrpt, ones1k):
    @functools.partial(
        pl.kernel,
        out_type=(jax.ShapeDtypeStruct((2 * NPAD,), jnp.float32),
                  jax.ShapeDtypeStruct((2 * CNT_BINS,), jnp.float32)),
        mesh=_sc_mesh(),
        compiler_params=pltpu.CompilerParams(use_tc_tiling_on_sc=False),
        scratch_types=[
            pltpu.VMEM_SHARED((NPAD,), jnp.float32),
            pltpu.VMEM_SHARED((CNT_BINS,), jnp.float32),
            pltpu.VMEM((1024,), jnp.int32),
            pltpu.VMEM((16, 128), jnp.int32),
            pltpu.VMEM((1024,), jnp.float32),
        ],
    )
    def k(dst_hbm, batch_hbm, z_hbm, ones_hbm, deg_out, cnt_out,
          sh_deg, sh_cnt, didx_v, bidx_v, ones_v):
        c = lax.axis_index("c")
        s = lax.axis_index("s")
        pltpu.sync_copy(ones_hbm, ones_v)
        pltpu.sync_copy(z_hbm, sh_deg.at[pl.ds(s * RPT, RPT)])
        pltpu.sync_copy(z_hbm.at[pl.ds(0, CNT_PT)],
                        sh_cnt.at[pl.ds(s * CNT_PT, CNT_PT)])
        ebase = c * 409600 + s * 25600
        w = c * 16 + s
        nj = 8 + 8 * (w < 18).astype(jnp.int32)

        @pl.when(w < 18)
        def _():
            pltpu.sync_copy(batch_hbm.at[pl.ds(w * 16, 16)], bidx_v)

        @pl.when(w >= 18)
        def _():
            pltpu.sync_copy(batch_hbm.at[pl.ds(288 + (w - 18) * 8, 8)],
                            bidx_v.at[pl.ds(0, 8)])

        plsc.subcore_barrier()

        @pl.loop(0, 25)
        def _(j):
            pltpu.sync_copy(dst_hbm.at[pl.ds(ebase + j * 1024, 1024)],
                            didx_v)
            pltpu.sync_copy(ones_v, sh_deg.at[didx_v], add=True)

        @pl.loop(0, nj)
        def _(j):
            pltpu.sync_copy(ones_v.at[pl.ds(0, 128)],
                            sh_cnt.at[bidx_v.at[j]], add=True)

        plsc.subcore_barrier()
        pltpu.sync_copy(sh_deg.at[pl.ds(s * RPT, RPT)],
                        deg_out.at[pl.ds(c * NPAD + s * RPT, RPT)])
        pltpu.sync_copy(sh_cnt.at[pl.ds(s * CNT_PT, CNT_PT)],
                        cnt_out.at[pl.ds(c * CNT_BINS + s * CNT_PT, CNT_PT)])

    return k(dst_flat, batch2d, zrpt, ones1k)


def _sc_prop(ms, src_flat, dst_flat, zinit, CT, NBUF=4):
    CPH = CT // 2
    split_edges = CT == 1
    ROWS = 200 if split_edges else 400
    SBLK = 40
    NSB = ROWS // SBLK

    @functools.partial(
        pl.kernel,
        out_type=jax.ShapeDtypeStruct((2 if split_edges else CT, NPAD, 32),
                                      jnp.float32),
        mesh=_sc_mesh(),
        compiler_params=pltpu.CompilerParams(use_tc_tiling_on_sc=False),
        scratch_types=[
            pltpu.VMEM_SHARED((NPAD, 32), jnp.float32),
            pltpu.VMEM((SBLK, 128), jnp.int32),
            pltpu.VMEM((SBLK, 128), jnp.int32),
        ] + [pltpu.VMEM((128, 32), jnp.float32)] * NBUF
          + [pltpu.SemaphoreType.DMA] * (2 * NBUF),
    )
    def k(ms_hbm, src_hbm, dst_hbm, z_hbm, out_hbm, sh, srcv, dstv, *rest):
        bufs = rest[:NBUF]
        gsems = rest[NBUF:2 * NBUF]
        ssems = rest[2 * NBUF:3 * NBUF]
        c = lax.axis_index("c")
        s = lax.axis_index("s")
        if split_edges:
            rbase = c * 3200 + s * ROWS
        else:
            rbase = s * ROWS

        def do_edges(src2d):
            for sb in range(NSB):
                pltpu.sync_copy(
                    src_hbm.at[pl.ds(rbase + sb * SBLK, SBLK)], srcv)
                pltpu.sync_copy(
                    dst_hbm.at[pl.ds(rbase + sb * SBLK, SBLK)], dstv)
                for b in range(2):
                    pltpu.async_copy(src2d.at[srcv.at[b]], bufs[b], gsems[b])

                @pl.loop(0, SBLK, step=NBUF)
                def _(j):
                    for b in range(NBUF):
                        jj = j + b
                        b2 = (b + 2) % NBUF
                        pltpu.make_async_copy(
                            src2d.at[srcv.at[jj]], bufs[b], gsems[b]).wait()
                        pltpu.async_copy(bufs[b], sh.at[dstv.at[jj]],
                                         ssems[b], add=True)
                        @pl.when(jj >= 2)
                        def _():
                            pltpu.make_async_copy(
                                bufs[b2], sh.at[dstv.at[jj]],
                                ssems[b2]).wait()
                        @pl.when(jj + 2 < SBLK)
                        def _():
                            pltpu.async_copy(src2d.at[srcv.at[jj + 2]],
                                             bufs[b2], gsems[b2])

                for b in ((SBLK - 2) % NBUF, (SBLK - 1) % NBUF):
                    pltpu.make_async_copy(
                        bufs[b], sh.at[dstv.at[b]], ssems[b]).wait()
            plsc.subcore_barrier()

        if split_edges:
            @pl.when(c == 0)
            def _():
                pltpu.sync_copy(ms_hbm.at[pl.ds(s * RPT, RPT)],
                                sh.at[pl.ds(s * RPT, RPT)])

            @pl.when(c == 1)
            def _():
                pltpu.sync_copy(z_hbm, sh.at[pl.ds(s * RPT, RPT)])

            plsc.subcore_barrier()
            do_edges(ms_hbm)
            pltpu.sync_copy(sh.at[pl.ds(s * RPT, RPT)],
                            out_hbm.at[c].at[pl.ds(s * RPT, RPT)])
        else:
            def do_chunk(ci):
                pltpu.sync_copy(ms_hbm.at[ci].at[pl.ds(s * RPT, RPT)],
                                sh.at[pl.ds(s * RPT, RPT)])
                plsc.subcore_barrier()
                do_edges(ms_hbm.at[ci])
                pltpu.sync_copy(sh.at[pl.ds(s * RPT, RPT)],
                                out_hbm.at[ci].at[pl.ds(s * RPT, RPT)])

            @pl.when(c == 0)
            def _():
                for kk in range(CPH):
                    do_chunk(kk)

            @pl.when(c == 1)
            def _():
                for kk in range(CPH):
                    do_chunk(CPH + kk)

    return k(ms, src_flat, dst_flat, zinit)


def _sc_pool(gx, batch2d, zpool):
    @functools.partial(
        pl.kernel,
        out_type=jax.ShapeDtypeStruct((2, POOL_ROWS, 128), jnp.float32),
        mesh=_sc_mesh(),
        compiler_params=pltpu.CompilerParams(use_tc_tiling_on_sc=False),
        scratch_types=[
            pltpu.VMEM_SHARED((POOL_ROWS, 128), jnp.float32),
            pltpu.VMEM((16, 128), jnp.int32),
            pltpu.VMEM((128, 128), jnp.float32),
        ],
    )
    def k(gx_hbm, batch_hbm, z_hbm, out_hbm, sh, bidx, buf):
        c = lax.axis_index("c")
        s = lax.axis_index("s")
        w = c * 16 + s
        base_blk = jnp.where(w < 18, w * 16, 288 + (w - 18) * 8)
        nj = 8 + 8 * (w < 18).astype(jnp.int32)

        @pl.when(w < 18)
        def _():
            pltpu.sync_copy(batch_hbm.at[pl.ds(w * 16, 16)], bidx)

        @pl.when(w >= 18)
        def _():
            pltpu.sync_copy(batch_hbm.at[pl.ds(288 + (w - 18) * 8, 8)],
                            bidx.at[pl.ds(0, 8)])

        pltpu.sync_copy(z_hbm, sh.at[pl.ds(s * POOL_PT, POOL_PT)])
        plsc.subcore_barrier()

        @pl.loop(0, nj)
        def _(j):
            pltpu.sync_copy(gx_hbm.at[pl.ds((base_blk + j) * 128, 128)], buf)
            pltpu.sync_copy(buf, sh.at[bidx.at[j]], add=True)

        plsc.subcore_barrier()
        pltpu.sync_copy(sh.at[pl.ds(s * POOL_PT, POOL_PT)],
                        out_hbm.at[c].at[pl.ds(s * POOL_PT, POOL_PT)])

    return k(gx, batch2d, zpool)


def _row_mask(i, rows):
    rid = lax.broadcasted_iota(jnp.int32, (rows, 1), 0) + i * rows
    return (rid < N).astype(jnp.float32)


def _tc1_body(x_ref, dp_ref, ms_ref, dis_ref):
    deg = dp_ref[0] + dp_ref[1] + 1.0
    dis = lax.rsqrt(deg)
    ms_ref[...] = x_ref[...] * dis
    dis_ref[...] = dis


def _tc1(xpad, deg_part):
    return pl.pallas_call(
        _tc1_body,
        grid=(GRID,),
        in_specs=[pl.BlockSpec((RB, 32), lambda i: (i, 0)),
                  pl.BlockSpec((2, RB, 1), lambda i: (0, i, 0))],
        out_specs=[pl.BlockSpec((RB, 32), lambda i: (i, 0)),
                   pl.BlockSpec((RB, 1), lambda i: (i, 0))],
        out_shape=[jax.ShapeDtypeStruct((NPAD, 32), jnp.float32),
                   jax.ShapeDtypeStruct((NPAD, 1), jnp.float32)],
    )(xpad, deg_part)


def _tc2a_body(p1_ref, dis_ref, w_ref, b_ref, a1_ref, st_ref):
    i = pl.program_id(0)
    prop = (p1_ref[0] + p1_ref[1]) * dis_ref[...]
    a1 = jnp.dot(prop, w_ref[...], preferred_element_type=jnp.float32,
                 precision=PREC) + b_ref[...][None, :]
    a1_ref[...] = a1
    am = a1 * _row_mask(i, RB)
    s1 = jnp.sum(am, axis=0)
    s2 = jnp.sum(am * am, axis=0)

    @pl.when(i == 0)
    def _():
        st_ref[...] = jnp.zeros_like(st_ref)

    st_ref[...] += jnp.concatenate(
        [s1[None, :], s2[None, :], jnp.zeros((6, s1.shape[0]), jnp.float32)], axis=0)


def _tc2a(p1, dis2d, Wg1, bg1):
    return pl.pallas_call(
        _tc2a_body,
        grid=(GRID,),
        in_specs=[pl.BlockSpec((2, RB, 32), lambda i: (0, i, 0)),
                  pl.BlockSpec((RB, 1), lambda i: (i, 0)),
                  pl.BlockSpec((32, 256), lambda i: (0, 0)),
                  pl.BlockSpec((256,), lambda i: (0,))],
        out_specs=[pl.BlockSpec((RB, 256), lambda i: (i, 0)),
                   pl.BlockSpec((8, 256), lambda i: (0, 0))],
        out_shape=[jax.ShapeDtypeStruct((NPAD, 256), jnp.float32),
                   jax.ShapeDtypeStruct((8, 256), jnp.float32)],
    )(p1, dis2d, Wg1, bg1)


def _tc2b_body(a1_ref, st_ref, g_ref, bta_ref, w2_ref, dis_ref, ms2_ref):
    i = pl.program_id(0)
    st = st_ref[...]
    mu = st[0, :] / N
    var = st[1, :] / N - mu * mu
    inv = lax.rsqrt(var + EPS) * g_ref[...]
    h = jnp.maximum((a1_ref[...] - mu[None, :]) * inv[None, :]
                    + bta_ref[...][None, :], 0.0)
    m2 = jnp.dot(h, w2_ref[...], preferred_element_type=jnp.float32,
                 precision=PREC)
    ms2 = m2 * dis_ref[...] * _row_mask(i, RB)
    ms2_ref[...] = jnp.stack(
        [ms2[:, 32 * k:32 * (k + 1)] for k in range(4)], axis=0)


def _tc2b(a1, stats1, gamma1, beta1, Wg2, dis2d):
    return pl.pallas_call(
        _tc2b_body,
        grid=(GRID,),
        in_specs=[pl.BlockSpec((RB, 256), lambda i: (i, 0)),
                  pl.BlockSpec((8, 256), lambda i: (0, 0)),
                  pl.BlockSpec((256,), lambda i: (0,)),
                  pl.BlockSpec((256,), lambda i: (0,)),
                  pl.BlockSpec((256, 128), lambda i: (0, 0)),
                  pl.BlockSpec((RB, 1), lambda i: (i, 0))],
        out_specs=pl.BlockSpec((4, RB, 32), lambda i: (0, i, 0)),
        out_shape=jax.ShapeDtypeStruct((4, NPAD, 32), jnp.float32),
    )(a1, stats1, gamma1, beta1, Wg2, dis2d)


def _tc3a_body(p2_ref, dis_ref, b_ref, a2_ref, st_ref):
    i = pl.program_id(0)
    a2 = jnp.concatenate([p2_ref[k] for k in range(4)],
                         axis=1) * dis_ref[...] + b_ref[...][None, :]
    a2_ref[...] = a2
    am = a2 * _row_mask(i, RB)
    s1 = jnp.sum(am, axis=0)
    s2 = jnp.sum(am * am, axis=0)

    @pl.when(i == 0)
    def _():
        st_ref[...] = jnp.zeros_like(st_ref)

    st_ref[...] += jnp.concatenate(
        [s1[None, :], s2[None, :], jnp.zeros((6, s1.shape[0]), jnp.float32)], axis=0)


def _tc3a(p2, dis2d, bg2):
    return pl.pallas_call(
        _tc3a_body,
        grid=(GRID,),
        in_specs=[pl.BlockSpec((4, RB, 32), lambda i: (0, i, 0)),
                  pl.BlockSpec((RB, 1), lambda i: (i, 0)),
                  pl.BlockSpec((128,), lambda i: (0,))],
        out_specs=[pl.BlockSpec((RB, 128), lambda i: (i, 0)),
                   pl.BlockSpec((8, 128), lambda i: (0, 0))],
        out_shape=[jax.ShapeDtypeStruct((NPAD, 128), jnp.float32),
                   jax.ShapeDtypeStruct((8, 128), jnp.float32)],
    )(p2, dis2d, bg2)


def _tc3b_body(a2_ref, st_ref, g_ref, bta_ref, gx_ref):
    i = pl.program_id(0)
    st = st_ref[...]
    mu = st[0, :] / N
    var = st[1, :] / N - mu * mu
    inv = lax.rsqrt(var + EPS) * g_ref[...]
    gx = jnp.maximum((a2_ref[...] - mu[None, :]) * inv[None, :]
                     + bta_ref[...][None, :], 0.0)
    gx_ref[...] = gx * _row_mask(i, RB)


def _tc3b(a2, stats2, gamma2, beta2):
    return pl.pallas_call(
        _tc3b_body,
        grid=(GRID,),
        in_specs=[pl.BlockSpec((RB, 128), lambda i: (i, 0)),
                  pl.BlockSpec((8, 128), lambda i: (0, 0)),
                  pl.BlockSpec((128,), lambda i: (0,)),
                  pl.BlockSpec((128,), lambda i: (0,))],
        out_specs=pl.BlockSpec((RB, 128), lambda i: (i, 0)),
        out_shape=jax.ShapeDtypeStruct((NPAD, 128), jnp.float32),
    )(a2, stats2, gamma2, beta2)


def _tc_codetab_body(cd_ref, td_ref, wc1, bc1, wc2, bc2, wc3, bc3,
                     wt1, bt1, wt2, bt2, co_ref, to_ref):
    def mm(a, w, b):
        return jnp.dot(a, w[...], preferred_element_type=jnp.float32,
                       precision=PREC) + b[...][None, :]

    h = jnp.maximum(mm(cd_ref[...], wc1, bc1), 0.0)
    h = jnp.maximum(mm(h, wc2, bc2), 0.0)
    co_ref[...] = mm(h, wc3, bc3)
    t = jnp.maximum(mm(td_ref[...], wt1, bt1), 0.0)
    to_ref[...] = jnp.maximum(mm(t, wt2, bt2), 0.0)


def _tc_codetab(code_data, tabular_data, Wc1, bc1, Wc2, bc2, Wc3, bc3,
                Wt1, bt1, Wt2, bt2):
    return pl.pallas_call(
        _tc_codetab_body,
        out_shape=[jax.ShapeDtypeStruct((B, 32), jnp.float32),
                   jax.ShapeDtypeStruct((B, 32), jnp.float32)],
    )(code_data, tabular_data, Wc1, bc1, Wc2, bc2, Wc3, bc3,
      Wt1, bt1, Wt2, bt2)


def _tc4_body(co_ref, to_ref, pp_ref, cp_ref, mw_ref,
              wf1, bf1, wf2, bf2, wf3, bf3, out_ref):
    mwrow = mw_ref[0, :]
    lane = lax.broadcasted_iota(jnp.int32, (128,), 0)
    valid = lane < 3
    mx = jnp.max(jnp.where(valid, mwrow, -jnp.inf))
    e = jnp.where(valid, jnp.exp(mwrow - mx), 0.0)
    wts = e / jnp.sum(e)
    cnt = cp_ref[0, :B] + cp_ref[1, :B]
    pooled = (pp_ref[0, :B, :] + pp_ref[1, :B, :]) \
        / jnp.maximum(cnt, 1.0)[:, None]
    fused = jnp.concatenate(
        [co_ref[...] * wts[0], to_ref[...] * wts[1], pooled * wts[2]], axis=1)

    def mm(a, w, b):
        return jnp.dot(a, w[...], preferred_element_type=jnp.float32,
                       precision=PREC) + b[...][None, :]

    o = jnp.maximum(mm(fused, wf1, bf1), 0.0)
    o = jnp.maximum(mm(o, wf2, bf2), 0.0)
    out_ref[...] = mm(o, wf3, bf3)


def _tc4(code_out, tab_out, pool_part, cnt_part, mwp,
         Wf1, bf1, Wf2, bf2, Wf3, bf3):
    return pl.pallas_call(
        _tc4_body,
        out_shape=jax.ShapeDtypeStruct((B, 4), jnp.float32),
    )(code_out, tab_out, pool_part, cnt_part, mwp,
      Wf1, bf1, Wf2, bf2, Wf3, bf3)


def kernel(code_data, tabular_data, x, edge_index, batch,
           Wc1, bc1, Wc2, bc2, Wc3, bc3,
           Wt1, bt1, Wt2, bt2,
           Wg1, bg1, gamma1, beta1, Wg2, bg2, gamma2, beta2,
           mw, Wf1, bf1, Wf2, bf2, Wf3, bf3):
    epad = jnp.full((EPAD - E,), PADN, jnp.int32)
    src_flat = jnp.concatenate([edge_index[0], epad])
    dst_flat = jnp.concatenate([edge_index[1], epad])
    src2d = src_flat.reshape(EPAD // 128, 128)
    dst2d = dst_flat.reshape(EPAD // 128, 128)
    batch2d = jnp.concatenate(
        [batch, jnp.full((NPAD - N,), B, jnp.int32)]).reshape(NBLK, 128)
    xpad = jnp.pad(x, ((0, NPAD - N), (0, 0)))
    zrpt = jnp.zeros((RPT,), jnp.float32)
    ones1k = jnp.ones((1024,), jnp.float32)
    zinit32 = jnp.zeros((RPT, 32), jnp.float32)
    zpool = jnp.zeros((POOL_PT, 128), jnp.float32)
    mwp = jnp.zeros((1, 128), jnp.float32).at[0, :3].set(mw)

    code_out, tab_out = _tc_codetab(code_data, tabular_data,
                                    Wc1, bc1, Wc2, bc2, Wc3, bc3,
                                    Wt1, bt1, Wt2, bt2)

    deg_flat, cnt_flat = _sc_deg_cnt(dst_flat, batch2d, zrpt, ones1k)
    deg_part = deg_flat.reshape(2, NPAD, 1)
    cnt_part = cnt_flat.reshape(2, CNT_BINS)
    ms1, dis2d = _tc1(xpad, deg_part)
    p1 = _sc_prop(ms1, src2d, dst2d, zinit32, CT=1)
    a1, stats1 = _tc2a(p1, dis2d, Wg1, bg1)
    ms2 = _tc2b(a1, stats1, gamma1, beta1, Wg2, dis2d)
    p2 = _sc_prop(ms2, src2d, dst2d, zinit32, CT=4)
    a2, stats2 = _tc3a(p2, dis2d, bg2)
    gx = _tc3b(a2, stats2, gamma2, beta2)
    pool_part = _sc_pool(gx, batch2d, zpool)

    return _tc4(code_out, tab_out, pool_part, cnt_part, mwp,
                Wf1, bf1, Wf2, bf2, Wf3, bf3)


# speedup vs baseline: 1.2923x; 1.0142x over previous
import functools

import jax
import jax.numpy as jnp
from jax import lax
from jax.experimental import pallas as pl
from jax.experimental.pallas import tpu as pltpu
from jax.experimental.pallas import tpu_sc as plsc

N = 50000
NPAD = 51200
RPT = NPAD // 16
E = 800000
EPAD = 819200
PADN = 50016
NBLK = 400
B = 1024
CNT_BINS = 2048
CNT_PT = CNT_BINS // 16
POOL_ROWS = 1152
POOL_PT = POOL_ROWS // 16

RB = 3200
D2R = RB // 128
GRID = 16
EPS = 1e-5
PREC = jax.lax.Precision.DEFAULT

@functools.lru_cache(maxsize=None)
def _sc_mesh():
    return plsc.VectorSubcoreMesh(core_axis_name="c", subcore_axis_name="s",
                                  num_cores=2, num_subcores=16)


def _sc_deg_cnt(dst_flat, batch2d, zrpt, ones1k):
    @functools.partial(
        pl.kernel,
        out_type=(jax.ShapeDtypeStruct((2 * NPAD,), jnp.float32),
                  jax.ShapeDtypeStruct((2 * CNT_BINS,), jnp.float32)),
        mesh=_sc_mesh(),
        compiler_params=pltpu.CompilerParams(use_tc_tiling_on_sc=False),
        scratch_types=[
            pltpu.VMEM_SHARED((NPAD,), jnp.float32),
            pltpu.VMEM_SHARED((CNT_BINS,), jnp.float32),
            pltpu.VMEM((1024,), jnp.int32),
            pltpu.VMEM((16, 128), jnp.int32),
            pltpu.VMEM((1024,), jnp.float32),
        ],
    )
    def k(dst_hbm, batch_hbm, z_hbm, ones_hbm, deg_out, cnt_out,
          sh_deg, sh_cnt, didx_v, bidx_v, ones_v):
        c = lax.axis_index("c")
        s = lax.axis_index("s")
        pltpu.sync_copy(ones_hbm, ones_v)
        pltpu.sync_copy(z_hbm, sh_deg.at[pl.ds(s * RPT, RPT)])
        pltpu.sync_copy(z_hbm.at[pl.ds(0, CNT_PT)],
                        sh_cnt.at[pl.ds(s * CNT_PT, CNT_PT)])
        ebase = c * 409600 + s * 25600
        w = c * 16 + s
        nj = 8 + 8 * (w < 18).astype(jnp.int32)

        @pl.when(w < 18)
        def _():
            pltpu.sync_copy(batch_hbm.at[pl.ds(w * 16, 16)], bidx_v)

        @pl.when(w >= 18)
        def _():
            pltpu.sync_copy(batch_hbm.at[pl.ds(288 + (w - 18) * 8, 8)],
                            bidx_v.at[pl.ds(0, 8)])

        plsc.subcore_barrier()

        @pl.loop(0, 25)
        def _(j):
            pltpu.sync_copy(dst_hbm.at[pl.ds(ebase + j * 1024, 1024)],
                            didx_v)
            pltpu.sync_copy(ones_v, sh_deg.at[didx_v], add=True)

        @pl.loop(0, nj)
        def _(j):
            pltpu.sync_copy(ones_v.at[pl.ds(0, 128)],
                            sh_cnt.at[bidx_v.at[j]], add=True)

        plsc.subcore_barrier()
        pltpu.sync_copy(sh_deg.at[pl.ds(s * RPT, RPT)],
                        deg_out.at[pl.ds(c * NPAD + s * RPT, RPT)])
        pltpu.sync_copy(sh_cnt.at[pl.ds(s * CNT_PT, CNT_PT)],
                        cnt_out.at[pl.ds(c * CNT_BINS + s * CNT_PT, CNT_PT)])

    return k(dst_flat, batch2d, zrpt, ones1k)


def _sc_prop(ms, src_flat, dst_flat, zinit, CT, NBUF=4):
    CPH = CT // 2
    split_edges = CT == 1
    ROWS = 200 if split_edges else 400
    SBLK = 40
    NSB = ROWS // SBLK

    @functools.partial(
        pl.kernel,
        out_type=jax.ShapeDtypeStruct((2 if split_edges else CT, NPAD, 32),
                                      jnp.float32),
        mesh=_sc_mesh(),
        compiler_params=pltpu.CompilerParams(use_tc_tiling_on_sc=False),
        scratch_types=[
            pltpu.VMEM_SHARED((NPAD, 32), jnp.float32),
            pltpu.VMEM((SBLK, 128), jnp.int32),
            pltpu.VMEM((SBLK, 128), jnp.int32),
        ] + [pltpu.VMEM((128, 32), jnp.float32)] * NBUF
          + [pltpu.SemaphoreType.DMA] * (2 * NBUF),
    )
    def k(ms_hbm, src_hbm, dst_hbm, z_hbm, out_hbm, sh, srcv, dstv, *rest):
        bufs = rest[:NBUF]
        gsems = rest[NBUF:2 * NBUF]
        ssems = rest[2 * NBUF:3 * NBUF]
        c = lax.axis_index("c")
        s = lax.axis_index("s")
        if split_edges:
            rbase = c * 3200 + s * ROWS
        else:
            rbase = s * ROWS

        def do_edges(src2d):
            for sb in range(NSB):
                pltpu.sync_copy(
                    src_hbm.at[pl.ds(rbase + sb * SBLK, SBLK)], srcv)
                pltpu.sync_copy(
                    dst_hbm.at[pl.ds(rbase + sb * SBLK, SBLK)], dstv)
                for b in range(2):
                    pltpu.async_copy(src2d.at[srcv.at[b]], bufs[b], gsems[b])

                @pl.loop(0, SBLK, step=NBUF)
                def _(j):
                    for b in range(NBUF):
                        jj = j + b
                        b2 = (b + 2) % NBUF
                        pltpu.make_async_copy(
                            src2d.at[srcv.at[jj]], bufs[b], gsems[b]).wait()
                        pltpu.async_copy(bufs[b], sh.at[dstv.at[jj]],
                                         ssems[b], add=True)
                        @pl.when(jj >= 2)
                        def _():
                            pltpu.make_async_copy(
                                bufs[b2], sh.at[dstv.at[jj]],
                                ssems[b2]).wait()
                        @pl.when(jj + 2 < SBLK)
                        def _():
                            pltpu.async_copy(src2d.at[srcv.at[jj + 2]],
                                             bufs[b2], gsems[b2])

                for b in ((SBLK - 2) % NBUF, (SBLK - 1) % NBUF):
                    pltpu.make_async_copy(
                        bufs[b], sh.at[dstv.at[b]], ssems[b]).wait()
            plsc.subcore_barrier()

        if split_edges:
            @pl.when(c == 0)
            def _():
                pltpu.sync_copy(ms_hbm.at[pl.ds(s * RPT, RPT)],
                                sh.at[pl.ds(s * RPT, RPT)])

            @pl.when(c == 1)
            def _():
                pltpu.sync_copy(z_hbm, sh.at[pl.ds(s * RPT, RPT)])

            plsc.subcore_barrier()
            do_edges(ms_hbm)
            pltpu.sync_copy(sh.at[pl.ds(s * RPT, RPT)],
                            out_hbm.at[c].at[pl.ds(s * RPT, RPT)])
        else:
            def do_chunk(ci):
                pltpu.sync_copy(ms_hbm.at[ci].at[pl.ds(s * RPT, RPT)],
                                sh.at[pl.ds(s * RPT, RPT)])
                plsc.subcore_barrier()
                do_edges(ms_hbm.at[ci])
                pltpu.sync_copy(sh.at[pl.ds(s * RPT, RPT)],
                                out_hbm.at[ci].at[pl.ds(s * RPT, RPT)])

            @pl.when(c == 0)
            def _():
                for kk in range(CPH):
                    do_chunk(kk)

            @pl.when(c == 1)
            def _():
                for kk in range(CPH):
                    do_chunk(CPH + kk)

    return k(ms, src_flat, dst_flat, zinit)


def _sc_pool(gx, batch2d, zpool):
    @functools.partial(
        pl.kernel,
        out_type=jax.ShapeDtypeStruct((2, POOL_ROWS, 128), jnp.float32),
        mesh=_sc_mesh(),
        compiler_params=pltpu.CompilerParams(use_tc_tiling_on_sc=False),
        scratch_types=[
            pltpu.VMEM_SHARED((POOL_ROWS, 128), jnp.float32),
            pltpu.VMEM((16, 128), jnp.int32),
            pltpu.VMEM((128, 128), jnp.float32),
        ],
    )
    def k(gx_hbm, batch_hbm, z_hbm, out_hbm, sh, bidx, buf):
        c = lax.axis_index("c")
        s = lax.axis_index("s")
        w = c * 16 + s
        base_blk = jnp.where(w < 18, w * 16, 288 + (w - 18) * 8)
        nj = 8 + 8 * (w < 18).astype(jnp.int32)

        @pl.when(w < 18)
        def _():
            pltpu.sync_copy(batch_hbm.at[pl.ds(w * 16, 16)], bidx)

        @pl.when(w >= 18)
        def _():
            pltpu.sync_copy(batch_hbm.at[pl.ds(288 + (w - 18) * 8, 8)],
                            bidx.at[pl.ds(0, 8)])

        pltpu.sync_copy(z_hbm, sh.at[pl.ds(s * POOL_PT, POOL_PT)])
        plsc.subcore_barrier()

        @pl.loop(0, nj)
        def _(j):
            pltpu.sync_copy(gx_hbm.at[pl.ds((base_blk + j) * 128, 128)], buf)
            pltpu.sync_copy(buf, sh.at[bidx.at[j]], add=True)

        plsc.subcore_barrier()
        pltpu.sync_copy(sh.at[pl.ds(s * POOL_PT, POOL_PT)],
                        out_hbm.at[c].at[pl.ds(s * POOL_PT, POOL_PT)])

    return k(gx, batch2d, zpool)


def _row_mask(i, rows):
    rid = lax.broadcasted_iota(jnp.int32, (rows, 1), 0) + i * rows
    return (rid < N).astype(jnp.float32)


def _tc1_body(x_ref, dp_ref, ms_ref, dis_ref):
    deg = dp_ref[0] + dp_ref[1] + 1.0
    dis = lax.rsqrt(deg)
    ms_ref[...] = x_ref[...] * dis
    dis_ref[...] = dis


def _tc1(xpad, deg_part):
    return pl.pallas_call(
        _tc1_body,
        grid=(GRID,),
        in_specs=[pl.BlockSpec((RB, 32), lambda i: (i, 0)),
                  pl.BlockSpec((2, RB, 1), lambda i: (0, i, 0))],
        out_specs=[pl.BlockSpec((RB, 32), lambda i: (i, 0)),
                   pl.BlockSpec((RB, 1), lambda i: (i, 0))],
        out_shape=[jax.ShapeDtypeStruct((NPAD, 32), jnp.float32),
                   jax.ShapeDtypeStruct((NPAD, 1), jnp.float32)],
    )(xpad, deg_part)


def _tc2a_body(p1_ref, dis_ref, w_ref, b_ref, a1_ref, st_ref):
    i = pl.program_id(0)
    prop = (p1_ref[0] + p1_ref[1]) * dis_ref[...]
    a1 = jnp.dot(prop, w_ref[...], preferred_element_type=jnp.float32,
                 precision=PREC) + b_ref[...][None, :]
    a1_ref[...] = a1
    am = a1 * _row_mask(i, RB)
    s1 = jnp.sum(am, axis=0)
    s2 = jnp.sum(am * am, axis=0)

    @pl.when(i == 0)
    def _():
        st_ref[...] = jnp.zeros_like(st_ref)

    st_ref[...] += jnp.concatenate(
        [s1[None, :], s2[None, :], jnp.zeros((6, s1.shape[0]), jnp.float32)], axis=0)


def _tc2a(p1, dis2d, Wg1, bg1):
    return pl.pallas_call(
        _tc2a_body,
        grid=(GRID,),
        in_specs=[pl.BlockSpec((2, RB, 32), lambda i: (0, i, 0)),
                  pl.BlockSpec((RB, 1), lambda i: (i, 0)),
                  pl.BlockSpec((32, 256), lambda i: (0, 0)),
                  pl.BlockSpec((256,), lambda i: (0,))],
        out_specs=[pl.BlockSpec((RB, 256), lambda i: (i, 0)),
                   pl.BlockSpec((8, 256), lambda i: (0, 0))],
        out_shape=[jax.ShapeDtypeStruct((NPAD, 256), jnp.float32),
                   jax.ShapeDtypeStruct((8, 256), jnp.float32)],
    )(p1, dis2d, Wg1, bg1)


def _tc2b_body(a1_ref, st_ref, g_ref, bta_ref, w2_ref, dis_ref, ms2_ref):
    i = pl.program_id(0)
    st = st_ref[...]
    mu = st[0, :] / N
    var = st[1, :] / N - mu * mu
    inv = lax.rsqrt(var + EPS) * g_ref[...]
    h = jnp.maximum((a1_ref[...] - mu[None, :]) * inv[None, :]
                    + bta_ref[...][None, :], 0.0)
    m2 = jnp.dot(h, w2_ref[...], preferred_element_type=jnp.float32,
                 precision=PREC)
    ms2 = m2 * dis_ref[...] * _row_mask(i, RB)
    ms2_ref[...] = jnp.stack(
        [ms2[:, 32 * k:32 * (k + 1)] for k in range(4)], axis=0)


def _tc2b(a1, stats1, gamma1, beta1, Wg2, dis2d):
    return pl.pallas_call(
        _tc2b_body,
        grid=(GRID,),
        in_specs=[pl.BlockSpec((RB, 256), lambda i: (i, 0)),
                  pl.BlockSpec((8, 256), lambda i: (0, 0)),
                  pl.BlockSpec((256,), lambda i: (0,)),
                  pl.BlockSpec((256,), lambda i: (0,)),
                  pl.BlockSpec((256, 128), lambda i: (0, 0)),
                  pl.BlockSpec((RB, 1), lambda i: (i, 0))],
        out_specs=pl.BlockSpec((4, RB, 32), lambda i: (0, i, 0)),
        out_shape=jax.ShapeDtypeStruct((4, NPAD, 32), jnp.float32),
    )(a1, stats1, gamma1, beta1, Wg2, dis2d)


def _tc3a_body(p2_ref, dis_ref, b_ref, a2_ref, st_ref):
    i = pl.program_id(0)
    a2 = jnp.concatenate([p2_ref[k] for k in range(4)],
                         axis=1) * dis_ref[...] + b_ref[...][None, :]
    a2_ref[...] = a2
    am = a2 * _row_mask(i, RB)
    s1 = jnp.sum(am, axis=0)
    s2 = jnp.sum(am * am, axis=0)

    @pl.when(i == 0)
    def _():
        st_ref[...] = jnp.zeros_like(st_ref)

    st_ref[...] += jnp.concatenate(
        [s1[None, :], s2[None, :], jnp.zeros((6, s1.shape[0]), jnp.float32)], axis=0)


def _tc3a(p2, dis2d, bg2):
    return pl.pallas_call(
        _tc3a_body,
        grid=(GRID,),
        in_specs=[pl.BlockSpec((4, RB, 32), lambda i: (0, i, 0)),
                  pl.BlockSpec((RB, 1), lambda i: (i, 0)),
                  pl.BlockSpec((128,), lambda i: (0,))],
        out_specs=[pl.BlockSpec((RB, 128), lambda i: (i, 0)),
                   pl.BlockSpec((8, 128), lambda i: (0, 0))],
        out_shape=[jax.ShapeDtypeStruct((NPAD, 128), jnp.float32),
                   jax.ShapeDtypeStruct((8, 128), jnp.float32)],
    )(p2, dis2d, bg2)


def _tc3b_body(a2_ref, st_ref, g_ref, bta_ref, gx_ref):
    i = pl.program_id(0)
    st = st_ref[...]
    mu = st[0, :] / N
    var = st[1, :] / N - mu * mu
    inv = lax.rsqrt(var + EPS) * g_ref[...]
    gx = jnp.maximum((a2_ref[...] - mu[None, :]) * inv[None, :]
                     + bta_ref[...][None, :], 0.0)
    gx_ref[...] = gx * _row_mask(i, RB)


def _tc3b(a2, stats2, gamma2, beta2):
    return pl.pallas_call(
        _tc3b_body,
        grid=(GRID,),
        in_specs=[pl.BlockSpec((RB, 128), lambda i: (i, 0)),
                  pl.BlockSpec((8, 128), lambda i: (0, 0)),
                  pl.BlockSpec((128,), lambda i: (0,)),
                  pl.BlockSpec((128,), lambda i: (0,))],
        out_specs=pl.BlockSpec((RB, 128), lambda i: (i, 0)),
        out_shape=jax.ShapeDtypeStruct((NPAD, 128), jnp.float32),
    )(a2, stats2, gamma2, beta2)


def _tc_codetab_body(cd_ref, td_ref, wc1, bc1, wc2, bc2, wc3, bc3,
                     wt1, bt1, wt2, bt2, co_ref, to_ref):
    def mm(a, w, b):
        return jnp.dot(a, w[...], preferred_element_type=jnp.float32,
                       precision=PREC) + b[...][None, :]

    h = jnp.maximum(mm(cd_ref[...], wc1, bc1), 0.0)
    h = jnp.maximum(mm(h, wc2, bc2), 0.0)
    co_ref[...] = mm(h, wc3, bc3)
    t = jnp.maximum(mm(td_ref[...], wt1, bt1), 0.0)
    to_ref[...] = jnp.maximum(mm(t, wt2, bt2), 0.0)


def _tc_codetab(code_data, tabular_data, Wc1, bc1, Wc2, bc2, Wc3, bc3,
                Wt1, bt1, Wt2, bt2):
    return pl.pallas_call(
        _tc_codetab_body,
        out_shape=[jax.ShapeDtypeStruct((B, 32), jnp.float32),
                   jax.ShapeDtypeStruct((B, 32), jnp.float32)],
    )(code_data, tabular_data, Wc1, bc1, Wc2, bc2, Wc3, bc3,
      Wt1, bt1, Wt2, bt2)


def _tc4_body(co_ref, to_ref, pp_ref, cp_ref, mw_ref,
              wf1, bf1, wf2, bf2, wf3, bf3, out_ref):
    mwrow = mw_ref[0, :]
    lane = lax.broadcasted_iota(jnp.int32, (128,), 0)
    valid = lane < 3
    mx = jnp.max(jnp.where(valid, mwrow, -jnp.inf))
    e = jnp.where(valid, jnp.exp(mwrow - mx), 0.0)
    wts = e / jnp.sum(e)
    cnt = cp_ref[0, :B] + cp_ref[1, :B]
    pooled = (pp_ref[0, :B, :] + pp_ref[1, :B, :]) \
        / jnp.maximum(cnt, 1.0)[:, None]
    fused = jnp.concatenate(
        [co_ref[...] * wts[0], to_ref[...] * wts[1], pooled * wts[2]], axis=1)

    def mm(a, w, b):
        return jnp.dot(a, w[...], preferred_element_type=jnp.float32,
                       precision=PREC) + b[...][None, :]

    o = jnp.maximum(mm(fused, wf1, bf1), 0.0)
    o = jnp.maximum(mm(o, wf2, bf2), 0.0)
    out_ref[...] = mm(o, wf3, bf3)


def _tc4(code_out, tab_out, pool_part, cnt_part, mwp,
         Wf1, bf1, Wf2, bf2, Wf3, bf3):
    return pl.pallas_call(
        _tc4_body,
        out_shape=jax.ShapeDtypeStruct((B, 4), jnp.float32),
    )(code_out, tab_out, pool_part, cnt_part, mwp,
      Wf1, bf1, Wf2, bf2, Wf3, bf3)


def kernel(code_data, tabular_data, x, edge_index, batch,
           Wc1, bc1, Wc2, bc2, Wc3, bc3,
           Wt1, bt1, Wt2, bt2,
           Wg1, bg1, gamma1, beta1, Wg2, bg2, gamma2, beta2,
           mw, Wf1, bf1, Wf2, bf2, Wf3, bf3):
    epad = jnp.full((EPAD - E,), PADN, jnp.int32)
    src_flat = jnp.concatenate([edge_index[0], epad])
    dst_flat = jnp.concatenate([edge_index[1], epad])
    src2d = src_flat.reshape(EPAD // 128, 128)
    dst2d = dst_flat.reshape(EPAD // 128, 128)
    batch2d = jnp.concatenate(
        [batch, jnp.full((NPAD - N,), B, jnp.int32)]).reshape(NBLK, 128)
    xpad = jnp.pad(x, ((0, NPAD - N), (0, 0)))
    zrpt = jnp.zeros((RPT,), jnp.float32)
    ones1k = jnp.ones((1024,), jnp.float32)
    zinit32 = jnp.zeros((RPT, 32), jnp.float32)
    zpool = jnp.zeros((POOL_PT, 128), jnp.float32)
    mwp = jnp.zeros((1, 128), jnp.float32).at[0, :3].set(mw)

    code_out, tab_out = _tc_codetab(code_data, tabular_data,
                                    Wc1, bc1, Wc2, bc2, Wc3, bc3,
                                    Wt1, bt1, Wt2, bt2)

    deg_flat, cnt_flat = _sc_deg_cnt(dst_flat, batch2d, zrpt, ones1k)
    deg_part = deg_flat.reshape(2, NPAD, 1)
    cnt_part = cnt_flat.reshape(2, CNT_BINS)
    ms1, dis2d = _tc1(xpad, deg_part)
    p1 = _sc_prop(ms1, src2d, dst2d, zinit32, CT=1)
    a1, stats1 = _tc2a(p1, dis2d, Wg1, bg1)
    ms2 = _tc2b(a1, stats1, gamma1, beta1, Wg2, dis2d)
    p2 = _sc_prop(ms2, src2d, dst2d, zinit32, CT=4)
    a2, stats2 = _tc3a(p2, dis2d, bg2)
    gx = _tc3b(a2, stats2, gamma2, beta2)
    pool_part = _sc_pool(gx, batch2d, zpool)

    return _tc4(code_out, tab_out, pool_part, cnt_part, mwp,
                Wf1, bf1, Wf2, bf2, Wf3, bf3)
